# baseline (speedup 1.0000x reference)
import functools

import jax
import jax.numpy as jnp
from jax import lax
from jax.experimental import pallas as pl
from jax.experimental.pallas import tpu as pltpu
from jax.experimental.pallas import tpu_sc as plsc

TEXT_HIDDEN = 1024
HIDDEN = 1024

NC = 2
NS = 16
NW = NC * NS

CHUNK = 32
NBUF = 3


def _sc_gather_body(nchunks, table, idx_hbm, out_hbm, idx_v, rows, *sems):
    gsems, wsems = sems[:NBUF], sems[NBUF:]
    wid = lax.axis_index("s") * NC + lax.axis_index("c")
    pltpu.sync_copy(idx_hbm.at[wid], idx_v)
    g = [None] * nchunks
    w = [None] * nchunks

    def start_gather(c):
        g[c] = pltpu.async_copy(
            table.at[idx_v.at[c]], rows.at[c % NBUF], gsems[c % NBUF])

    def start_write(c):
        base = wid * (nchunks * CHUNK) + c * CHUNK
        w[c] = pltpu.async_copy(
            rows.at[c % NBUF], out_hbm.at[pl.ds(base, CHUNK)], wsems[c % NBUF])

    for c in range(min(NBUF, nchunks)):
        start_gather(c)
    for c in range(nchunks):
        g[c].wait()
        start_write(c)
        if c + NBUF < nchunks:
            w[c].wait()
            start_gather(c + NBUF)
    for c in range(max(0, nchunks - NBUF), nchunks):
        w[c].wait()


def _sc_gather(table, idx):
    n = idx.shape[0]
    d = table.shape[1]
    assert n % (NW * CHUNK) == 0
    nchunks = n // (NW * CHUNK)
    mesh = plsc.VectorSubcoreMesh(core_axis_name="c", subcore_axis_name="s")
    fn = pl.kernel(
        functools.partial(_sc_gather_body, nchunks),
        out_type=jax.ShapeDtypeStruct((n, d), jnp.float32),
        mesh=mesh,
        scratch_types=[
            pltpu.VMEM((nchunks, CHUNK), jnp.int32),
            pltpu.VMEM((NBUF, CHUNK, d), jnp.float32),
        ] + [pltpu.SemaphoreType.DMA] * (2 * NBUF),
    )
    return fn(table, idx.reshape(NW, nchunks, CHUNK))


def _mlp_block(x_ref, w1_ref, b1_ref, w2_ref, b2_ref, *rest):
    o_ref = rest[-1]
    x = x_ref[...].astype(jnp.bfloat16)
    h = jnp.dot(x, w1_ref[...].astype(jnp.bfloat16),
                preferred_element_type=jnp.float32)
    h = h + b1_ref[...]
    h = h * jax.nn.sigmoid(h)
    o = jnp.dot(h.astype(jnp.bfloat16), w2_ref[...].astype(jnp.bfloat16),
                preferred_element_type=jnp.float32)
    o_ref[...] = o + b2_ref[...]


def _tc_mlp_part(x, W1, b1, W2, b2, out_prev, row_offset, n_total, blk=2048):
    n = x.shape[0]
    grid = (n // blk,)
    off = row_offset // blk
    in_specs = [
        pl.BlockSpec((blk, TEXT_HIDDEN), lambda i: (i, 0)),
        pl.BlockSpec((TEXT_HIDDEN, TEXT_HIDDEN), lambda i: (0, 0)),
        pl.BlockSpec((1, TEXT_HIDDEN), lambda i: (0, 0)),
        pl.BlockSpec((TEXT_HIDDEN, HIDDEN), lambda i: (0, 0)),
        pl.BlockSpec((1, HIDDEN), lambda i: (0, 0)),
    ]
    args = [x, W1, b1.reshape(1, -1), W2, b2.reshape(1, -1)]
    aliases = {}
    if out_prev is not None:
        in_specs.append(pl.BlockSpec(memory_space=pl.ANY))
        args.append(out_prev)
        aliases = {5: 0}
    return pl.pallas_call(
        _mlp_block,
        grid=grid,
        in_specs=in_specs,
        out_specs=pl.BlockSpec((blk, HIDDEN), lambda i: (i + off, 0)),
        out_shape=jax.ShapeDtypeStruct((n_total, HIDDEN), jnp.float32),
        input_output_aliases=aliases,
    )(*args)


def kernel(text_table, W1, b1, W2, b2, codec_table, text_ids, codec_ids):
    B, T = text_ids.shape
    n = B * T
    cut = n // 2
    text_idx = text_ids.reshape(n).astype(jnp.int32)
    codec_idx = codec_ids.reshape(n).astype(jnp.int32)
    g_a = _sc_gather(text_table, text_idx[:cut])
    g_b = _sc_gather(text_table, text_idx[cut:])
    codec_embeds = _sc_gather(codec_table, codec_idx)
    o1 = _tc_mlp_part(g_a, W1, b1, W2, b2, None, 0, n)
    text_out = _tc_mlp_part(g_b, W1, b1, W2, b2, o1, cut, n)
    return (text_out.reshape(B, T, HIDDEN),
            codec_embeds.reshape(B, codec_ids.shape[1], HIDDEN))


# speedup vs baseline: 1.1343x; 1.1343x over previous
import functools

import jax
import jax.numpy as jnp
from jax import lax
from jax.experimental import pallas as pl
from jax.experimental.pallas import tpu as pltpu
from jax.experimental.pallas import tpu_sc as plsc

TEXT_HIDDEN = 1024
HIDDEN = 1024

NC = 2
NS = 16
NW = NC * NS

CHUNK = 32
NBUF = 3


def _sc_gather_body(nchunks, table, idx_hbm, out_hbm, idx_v, rows, *sems):
    gsems, wsems = sems[:NBUF], sems[NBUF:]
    wid = lax.axis_index("s") * NC + lax.axis_index("c")
    pltpu.sync_copy(idx_hbm.at[wid], idx_v)
    g = [None] * nchunks
    w = [None] * nchunks

    def start_gather(c):
        g[c] = pltpu.async_copy(
            table.at[idx_v.at[c]], rows.at[c % NBUF], gsems[c % NBUF])

    def start_write(c):
        base = wid * (nchunks * CHUNK) + c * CHUNK
        w[c] = pltpu.async_copy(
            rows.at[c % NBUF], out_hbm.at[pl.ds(base, CHUNK)], wsems[c % NBUF])

    for c in range(min(NBUF, nchunks)):
        start_gather(c)
    for c in range(nchunks):
        g[c].wait()
        start_write(c)
        if c + NBUF < nchunks:
            w[c].wait()
            start_gather(c + NBUF)
    for c in range(max(0, nchunks - NBUF), nchunks):
        w[c].wait()


def _sc_gather(table, idx):
    n = idx.shape[0]
    d = table.shape[1]
    assert n % (NW * CHUNK) == 0
    nchunks = n // (NW * CHUNK)
    mesh = plsc.VectorSubcoreMesh(core_axis_name="c", subcore_axis_name="s")
    fn = pl.kernel(
        functools.partial(_sc_gather_body, nchunks),
        out_type=jax.ShapeDtypeStruct((n, d), jnp.float32),
        mesh=mesh,
        scratch_types=[
            pltpu.VMEM((nchunks, CHUNK), jnp.int32),
            pltpu.VMEM((NBUF, CHUNK, d), jnp.float32),
        ] + [pltpu.SemaphoreType.DMA] * (2 * NBUF),
    )
    return fn(table, idx.reshape(NW, nchunks, CHUNK))


def _mlp_block(x_ref, w1_ref, b1_ref, w2_ref, b2_ref, *rest):
    o_ref = rest[-1]
    x = x_ref[...].astype(jnp.bfloat16)
    h = jnp.dot(x, w1_ref[...].astype(jnp.bfloat16),
                preferred_element_type=jnp.float32)
    h = h + b1_ref[...]
    h = h * jax.nn.sigmoid(h)
    o = jnp.dot(h.astype(jnp.bfloat16), w2_ref[...].astype(jnp.bfloat16),
                preferred_element_type=jnp.float32)
    o_ref[...] = o + b2_ref[...]


def _tc_mlp_part(x, W1, b1, W2, b2, out_prev, row_offset, n_total, blk=2048):
    n = x.shape[0]
    blk = min(blk, n)
    grid = (n // blk,)
    off = row_offset // blk
    in_specs = [
        pl.BlockSpec((blk, TEXT_HIDDEN), lambda i: (i, 0)),
        pl.BlockSpec((TEXT_HIDDEN, TEXT_HIDDEN), lambda i: (0, 0)),
        pl.BlockSpec((1, TEXT_HIDDEN), lambda i: (0, 0)),
        pl.BlockSpec((TEXT_HIDDEN, HIDDEN), lambda i: (0, 0)),
        pl.BlockSpec((1, HIDDEN), lambda i: (0, 0)),
    ]
    args = [x, W1, b1.reshape(1, -1), W2, b2.reshape(1, -1)]
    aliases = {}
    if out_prev is not None:
        in_specs.append(pl.BlockSpec(memory_space=pl.ANY))
        args.append(out_prev)
        aliases = {5: 0}
    return pl.pallas_call(
        _mlp_block,
        grid=grid,
        in_specs=in_specs,
        out_specs=pl.BlockSpec((blk, HIDDEN), lambda i: (i + off, 0)),
        out_shape=jax.ShapeDtypeStruct((n_total, HIDDEN), jnp.float32),
        input_output_aliases=aliases,
    )(*args)


def kernel(text_table, W1, b1, W2, b2, codec_table, text_ids, codec_ids):
    B, T = text_ids.shape
    n = B * T
    cut = n // 8
    text_idx = text_ids.reshape(n).astype(jnp.int32)
    codec_idx = codec_ids.reshape(n).astype(jnp.int32)
    g_a = _sc_gather(text_table, text_idx[:cut])
    g_b = _sc_gather(text_table, text_idx[cut:])
    codec_embeds = _sc_gather(codec_table, codec_idx)
    o1 = _tc_mlp_part(g_a, W1, b1, W2, b2, None, 0, n)
    text_out = _tc_mlp_part(g_b, W1, b1, W2, b2, o1, cut, n)
    return (text_out.reshape(B, T, HIDDEN),
            codec_embeds.reshape(B, codec_ids.shape[1], HIDDEN))
